# baseline (speedup 1.0000x reference)
import jax
import jax.numpy as jnp
from jax import lax
from jax.experimental import pallas as pl
from jax.experimental.pallas import tpu as pltpu
from jax.experimental.pallas import tpu_sc as plsc

VOCAB = 1000000
N_EMBD = 64
B, L = 4096, 200

NW = 32
BPW = B // NW
CH = 2
STAGES = BPW // CH
NBUF = 2
SPLITS = ((0, 128), (128, 72))


def _gather_body(x_hbm, table_hbm, out_hbm, idx_v, rows_v, gsems, osems):
    c = lax.axis_index("c")
    s = lax.axis_index("s")
    wid = s * 2 + c
    base = wid * BPW

    pltpu.sync_copy(x_hbm.at[pl.ds(base, BPW)], idx_v)

    def fire_gathers(b, stage):
        for i in range(CH):
            row = stage * CH + i
            for off, n in SPLITS:
                pltpu.async_copy(
                    table_hbm.at[idx_v.at[row, pl.ds(off, n)]],
                    rows_v.at[b, i, pl.ds(off, n)],
                    gsems[b],
                )

    def drain(sem, brow, b):
        pltpu.make_async_copy(
            out_hbm.at[pl.ds(brow, CH)], rows_v.at[b], sem
        ).wait()

    for b in range(NBUF):
        fire_gathers(b, b)

    @pl.loop(0, (STAGES - NBUF) // NBUF)
    def _t(t):
        for b in range(NBUF):
            stage = t * NBUF + b
            brow = base + stage * CH
            drain(gsems[b], brow, b)
            pltpu.async_copy(rows_v.at[b], out_hbm.at[pl.ds(brow, CH)], osems[b]).wait()
            fire_gathers(b, stage + NBUF)

    for b in range(NBUF):
        stage_e = STAGES - NBUF + b
        brow_e = base + stage_e * CH
        drain(gsems[b], brow_e, b)
        pltpu.async_copy(rows_v.at[b], out_hbm.at[pl.ds(brow_e, CH)], osems[b]).wait()


@jax.jit
def _embed_lookup(x, token_embed):
    mesh = plsc.VectorSubcoreMesh(core_axis_name="c", subcore_axis_name="s")
    return pl.kernel(
        _gather_body,
        out_type=jax.ShapeDtypeStruct((B, L, N_EMBD), jnp.float32),
        mesh=mesh,
        scratch_types=[
            pltpu.VMEM((BPW, L), jnp.int32),
            pltpu.VMEM((NBUF, CH, L, N_EMBD), jnp.float32),
            [pltpu.SemaphoreType.DMA] * NBUF,
            [pltpu.SemaphoreType.DMA] * NBUF,
        ],
        compiler_params=pltpu.CompilerParams(use_tc_tiling_on_sc=False),
    )(x, token_embed)


def kernel(x, token_embed):
    return _embed_lookup(x.astype(jnp.int32), token_embed)
